# baseline (speedup 1.0000x reference)
import functools

import jax
import jax.numpy as jnp
from jax.experimental import pallas as pl
from jax.experimental.pallas import tpu as pltpu

_VMEM_LIMIT = 100 * 1024 * 1024
_SLOTS = 3


def _interp_matrix(in_size, out_size):
    scale = in_size / out_size
    dst = jnp.arange(out_size, dtype=jnp.float32)
    src = jnp.maximum((dst + 0.5) * scale - 0.5, 0.0)
    x0f = jnp.floor(src)
    lam = src - x0f
    x0 = jnp.minimum(x0f.astype(jnp.int32), in_size - 1)
    x1 = jnp.minimum(x0 + 1, in_size - 1)
    cols = jnp.arange(in_size, dtype=jnp.int32)[None, :]
    return ((1.0 - lam)[:, None] * (cols == x0[:, None])
            + lam[:, None] * (cols == x1[:, None])).astype(jnp.float32)


def _fused_head_kernel(x_ref, wcomb_ref, bcomb_ref, wp_ref, bp_ref, qe_ref,
                       w1_ref, b1_ref, w2_ref, b2_ref, ah_ref, awt_ref, o_ref,
                       obuf_ref, sem_ref, *, mask_dim, num_q, h_in, w_in,
                       steps_per_core):
    hw = h_in * w_in
    k = pl.program_id(1)
    img = pl.program_id(0) * steps_per_core + k
    slot = jax.lax.rem(k, _SLOTS)

    def out_copy(slot_idx, img_idx):
        return pltpu.make_async_copy(
            obuf_ref.at[slot_idx], o_ref.at[img_idx], sem_ref.at[slot_idx])

    @pl.when(k >= _SLOTS)
    def _free_slot():
        out_copy(slot, img - _SLOTS).wait()

    x = x_ref[0]
    comb = jnp.maximum(
        jnp.dot(wcomb_ref[...], x, preferred_element_type=jnp.float32)
        + bcomb_ref[...], 0.0)
    mf = comb[:mask_dim]

    s = jnp.sum(comb[mask_dim:], axis=1, keepdims=True)
    pooled = (jnp.dot(jnp.transpose(s), wp_ref[...],
                      preferred_element_type=jnp.float32) * (1.0 / hw)
              + bp_ref[...])

    q = qe_ref[...] + pooled
    h = jnp.maximum(jnp.dot(q, w1_ref[...],
                            preferred_element_type=jnp.float32) + b1_ref[...], 0.0)
    e = jnp.dot(h, w2_ref[...],
                preferred_element_type=jnp.float32) + b2_ref[...]

    pred = jnp.dot(e, mf, preferred_element_type=jnp.float32)
    pred2 = pred.reshape(num_q * h_in, w_in)
    t = jnp.dot(pred2, awt_ref[...],
                preferred_element_type=jnp.float32)
    for qi in range(num_q):
        obuf_ref[slot, qi] = jnp.dot(ah_ref[...], t[qi * h_in:(qi + 1) * h_in],
                                     preferred_element_type=jnp.float32)
        pltpu.make_async_copy(obuf_ref.at[slot, qi], o_ref.at[img, qi],
                              sem_ref.at[slot]).start()

    @pl.when(k == steps_per_core - 1)
    def _drain():
        for j in range(_SLOTS - 1, -1, -1):
            out_copy(jax.lax.rem(k - j + _SLOTS, _SLOTS), img - j).wait()


def kernel(res2, wm_t, we_t, wp_t, pd_mask_b, pd_enc_b, enc_proj_b,
           query_embed, mlp_w1, mlp_b1, mlp_w2, mlp_b2):
    N, Cin, H, W = res2.shape
    HW = H * W
    mask_dim = wm_t.shape[0]
    conv_dim = we_t.shape[0]
    hidden = wp_t.shape[0]
    Q = query_embed.shape[0]
    stride = 4
    Ho, Wo = H * stride, W * stride
    n_cores = 2 if N % 2 == 0 else 1
    steps = N // n_cores

    x = res2.reshape(N, Cin, HW)
    wcomb = jnp.concatenate([wm_t, we_t], axis=0)
    bcomb = jnp.concatenate([pd_mask_b, pd_enc_b])[:, None]
    ah = _interp_matrix(H, Ho)
    awt = jnp.transpose(_interp_matrix(W, Wo))

    out = pl.pallas_call(
        functools.partial(_fused_head_kernel, mask_dim=mask_dim, num_q=Q,
                          h_in=H, w_in=W, steps_per_core=steps),
        out_shape=jax.ShapeDtypeStruct((N, Q, Ho, Wo), jnp.float32),
        grid=(n_cores, steps),
        in_specs=[
            pl.BlockSpec((1, Cin, HW), lambda c, k: (c * (N // 2) + k, 0, 0)
                         if N % 2 == 0 else (k, 0, 0)),
            pl.BlockSpec((mask_dim + conv_dim, Cin), lambda c, k: (0, 0)),
            pl.BlockSpec((mask_dim + conv_dim, 1), lambda c, k: (0, 0)),
            pl.BlockSpec((conv_dim, hidden), lambda c, k: (0, 0)),
            pl.BlockSpec((1, hidden), lambda c, k: (0, 0)),
            pl.BlockSpec((Q, hidden), lambda c, k: (0, 0)),
            pl.BlockSpec((hidden, hidden), lambda c, k: (0, 0)),
            pl.BlockSpec((1, hidden), lambda c, k: (0, 0)),
            pl.BlockSpec((hidden, mask_dim), lambda c, k: (0, 0)),
            pl.BlockSpec((1, mask_dim), lambda c, k: (0, 0)),
            pl.BlockSpec((Ho, H), lambda c, k: (0, 0)),
            pl.BlockSpec((W, Wo), lambda c, k: (0, 0)),
        ],
        out_specs=pl.BlockSpec(memory_space=pl.ANY),
        scratch_shapes=[pltpu.VMEM((_SLOTS, Q, Ho, Wo), jnp.float32),
                        pltpu.SemaphoreType.DMA((_SLOTS,))],
        compiler_params=pltpu.CompilerParams(
            dimension_semantics=("parallel", "arbitrary"),
            vmem_limit_bytes=_VMEM_LIMIT),
    )(x, wcomb, bcomb, jnp.transpose(wp_t),
      enc_proj_b[None, :], query_embed, mlp_w1, mlp_b1[None, :], mlp_w2,
      mlp_b2[None, :], ah, awt)
    return out


# speedup vs baseline: 1.0042x; 1.0042x over previous
import functools

import jax
import jax.numpy as jnp
from jax.experimental import pallas as pl
from jax.experimental.pallas import tpu as pltpu

_VMEM_LIMIT = 100 * 1024 * 1024
_SLOTS = 4


def _interp_matrix(in_size, out_size):
    scale = in_size / out_size
    dst = jnp.arange(out_size, dtype=jnp.float32)
    src = jnp.maximum((dst + 0.5) * scale - 0.5, 0.0)
    x0f = jnp.floor(src)
    lam = src - x0f
    x0 = jnp.minimum(x0f.astype(jnp.int32), in_size - 1)
    x1 = jnp.minimum(x0 + 1, in_size - 1)
    cols = jnp.arange(in_size, dtype=jnp.int32)[None, :]
    return ((1.0 - lam)[:, None] * (cols == x0[:, None])
            + lam[:, None] * (cols == x1[:, None])).astype(jnp.float32)


def _fused_head_kernel(x_ref, wcomb_ref, bcomb_ref, wp_ref, bp_ref, qe_ref,
                       w1_ref, b1_ref, w2_ref, b2_ref, ah_ref, awt_ref, o_ref,
                       obuf_ref, sem_ref, *, mask_dim, num_q, h_in, w_in,
                       steps_per_core):
    hw = h_in * w_in
    k = pl.program_id(1)
    img = pl.program_id(0) * steps_per_core + k
    slot = jax.lax.rem(k, _SLOTS)

    def out_copy(slot_idx, img_idx):
        return pltpu.make_async_copy(
            obuf_ref.at[slot_idx], o_ref.at[img_idx], sem_ref.at[slot_idx])

    @pl.when(k >= _SLOTS)
    def _free_slot():
        out_copy(slot, img - _SLOTS).wait()

    x = x_ref[0]
    comb = jnp.maximum(
        jnp.dot(wcomb_ref[...], x, preferred_element_type=jnp.float32)
        + bcomb_ref[...], 0.0)
    mf = comb[:mask_dim]

    s = jnp.sum(comb[mask_dim:], axis=1, keepdims=True)
    pooled = (jnp.dot(jnp.transpose(s), wp_ref[...],
                      preferred_element_type=jnp.float32) * (1.0 / hw)
              + bp_ref[...])

    q = qe_ref[...] + pooled
    h = jnp.maximum(jnp.dot(q, w1_ref[...],
                            preferred_element_type=jnp.float32) + b1_ref[...], 0.0)
    e = jnp.dot(h, w2_ref[...],
                preferred_element_type=jnp.float32) + b2_ref[...]

    pred = jnp.dot(e, mf, preferred_element_type=jnp.float32)
    pred2 = pred.reshape(num_q * h_in, w_in)
    t = jnp.dot(pred2, awt_ref[...],
                preferred_element_type=jnp.float32)
    for qi in range(num_q):
        obuf_ref[slot, qi] = jnp.dot(ah_ref[...], t[qi * h_in:(qi + 1) * h_in],
                                     preferred_element_type=jnp.float32)
        pltpu.make_async_copy(obuf_ref.at[slot, qi], o_ref.at[img, qi],
                              sem_ref.at[slot]).start()

    @pl.when(k == steps_per_core - 1)
    def _drain():
        for j in range(_SLOTS - 1, -1, -1):
            out_copy(jax.lax.rem(k - j + _SLOTS, _SLOTS), img - j).wait()


def kernel(res2, wm_t, we_t, wp_t, pd_mask_b, pd_enc_b, enc_proj_b,
           query_embed, mlp_w1, mlp_b1, mlp_w2, mlp_b2):
    N, Cin, H, W = res2.shape
    HW = H * W
    mask_dim = wm_t.shape[0]
    conv_dim = we_t.shape[0]
    hidden = wp_t.shape[0]
    Q = query_embed.shape[0]
    stride = 4
    Ho, Wo = H * stride, W * stride
    n_cores = 2 if N % 2 == 0 else 1
    steps = N // n_cores

    x = res2.reshape(N, Cin, HW)
    wcomb = jnp.concatenate([wm_t, we_t], axis=0)
    bcomb = jnp.concatenate([pd_mask_b, pd_enc_b])[:, None]
    ah = _interp_matrix(H, Ho)
    awt = jnp.transpose(_interp_matrix(W, Wo))

    out = pl.pallas_call(
        functools.partial(_fused_head_kernel, mask_dim=mask_dim, num_q=Q,
                          h_in=H, w_in=W, steps_per_core=steps),
        out_shape=jax.ShapeDtypeStruct((N, Q, Ho, Wo), jnp.float32),
        grid=(n_cores, steps),
        in_specs=[
            pl.BlockSpec((1, Cin, HW), lambda c, k: (c * (N // 2) + k, 0, 0)
                         if N % 2 == 0 else (k, 0, 0)),
            pl.BlockSpec((mask_dim + conv_dim, Cin), lambda c, k: (0, 0)),
            pl.BlockSpec((mask_dim + conv_dim, 1), lambda c, k: (0, 0)),
            pl.BlockSpec((conv_dim, hidden), lambda c, k: (0, 0)),
            pl.BlockSpec((1, hidden), lambda c, k: (0, 0)),
            pl.BlockSpec((Q, hidden), lambda c, k: (0, 0)),
            pl.BlockSpec((hidden, hidden), lambda c, k: (0, 0)),
            pl.BlockSpec((1, hidden), lambda c, k: (0, 0)),
            pl.BlockSpec((hidden, mask_dim), lambda c, k: (0, 0)),
            pl.BlockSpec((1, mask_dim), lambda c, k: (0, 0)),
            pl.BlockSpec((Ho, H), lambda c, k: (0, 0)),
            pl.BlockSpec((W, Wo), lambda c, k: (0, 0)),
        ],
        out_specs=pl.BlockSpec(memory_space=pl.ANY),
        scratch_shapes=[pltpu.VMEM((_SLOTS, Q, Ho, Wo), jnp.float32),
                        pltpu.SemaphoreType.DMA((_SLOTS,))],
        compiler_params=pltpu.CompilerParams(
            dimension_semantics=("parallel", "arbitrary"),
            vmem_limit_bytes=_VMEM_LIMIT),
    )(x, wcomb, bcomb, jnp.transpose(wp_t),
      enc_proj_b[None, :], query_embed, mlp_w1, mlp_b1[None, :], mlp_w2,
      mlp_b2[None, :], ah, awt)
    return out
